# Initial kernel scaffold; baseline (speedup 1.0000x reference)
import jax
import jax.numpy as jnp
from jax.experimental import pallas as pl


def kernel(q, xyz):
    raise NotImplementedError("write your pallas kernel here")


# speedup vs baseline: 1.3480x; 1.3480x over previous
import functools

import jax
import jax.numpy as jnp
from jax.experimental import pallas as pl
from jax.experimental.pallas import tpu as pltpu

KE_KCAL = 332.0716
R_ON = 4.0
R_OFF = 5.0

BI = 512
BJ = 512


def _tile_kernel(col_ref, row_ref, out_ref):
    bi = pl.program_id(0)
    bj = pl.program_id(1)

    colb = col_ref[...]
    rowb = row_ref[...]

    dx = colb[:, 0:1] - rowb[0:1, :]
    dy = colb[:, 1:2] - rowb[1:2, :]
    dz = colb[:, 2:3] - rowb[2:3, :]
    r2 = dx * dx + dy * dy + dz * dz
    qq = colb[:, 3:4] * rowb[3:4, :]

    ii = bi * BI + jax.lax.broadcasted_iota(jnp.int32, (BI, BJ), 0)
    jj = bj * BJ + jax.lax.broadcasted_iota(jnp.int32, (BI, BJ), 1)
    mask = (jj > ii) & (r2 > 0.0)

    r2s = jnp.where(mask, r2, 1.0)
    r = jnp.sqrt(r2s)
    a = jnp.clip(r - R_ON, 0.0, 1.0) * (1.0 / (R_OFF - R_ON))
    t = jnp.exp(1.0 / (1.0 - a) - 1.0 / a)
    fs = 1.0 / (1.0 + t)
    e = qq * (fs * jax.lax.rsqrt(r2s + 1.0) + (1.0 - fs) / r)
    esum = jnp.sum(jnp.where(mask, e, 0.0), axis=(0, 1), keepdims=True)

    @pl.when((bi == 0) & (bj == 0))
    def _():
        out_ref[...] = jnp.zeros_like(out_ref)

    out_ref[...] += KE_KCAL * esum


@functools.partial(jax.jit, static_argnames=())
def kernel(q, xyz):
    n = xyz.shape[0]
    row = jnp.concatenate([xyz.T, q[None, :], jnp.zeros((4, n), jnp.float32)], axis=0)
    col = row.T

    nbi = n // BI
    nbj = n // BJ
    out = pl.pallas_call(
        _tile_kernel,
        grid=(nbi, nbj),
        in_specs=[
            pl.BlockSpec((BI, 8), lambda i, j: (i, 0)),
            pl.BlockSpec((8, BJ), lambda i, j: (0, j)),
        ],
        out_specs=pl.BlockSpec((1, 1), lambda i, j: (0, 0)),
        out_shape=jax.ShapeDtypeStruct((1, 1), jnp.float32),
        compiler_params=pltpu.CompilerParams(
            dimension_semantics=("arbitrary", "arbitrary"),
        ),
    )(col, row)
    return out[0, 0]


# speedup vs baseline: 1.4827x; 1.0999x over previous
import functools

import jax
import jax.numpy as jnp
import numpy as np
from jax.experimental import pallas as pl
from jax.experimental.pallas import tpu as pltpu

KE_KCAL = 332.0716
R_ON = 4.0
R_OFF = 5.0

N = 2048
BI = 256
BJ = 256
_NB = N // BI
_BIS = np.array([i for i in range(_NB) for j in range(i, _NB)], dtype=np.int32)
_BJS = np.array([j for i in range(_NB) for j in range(i, _NB)], dtype=np.int32)
_T = len(_BIS)


def _tile_kernel(bis_ref, bjs_ref, col_ref, row_ref, out_ref):
    t = pl.program_id(0)
    bi = bis_ref[t]
    bj = bjs_ref[t]

    colb = col_ref[...]
    rowb = row_ref[...]

    dx = colb[:, 0:1] - rowb[0:1, :]
    dy = colb[:, 1:2] - rowb[1:2, :]
    dz = colb[:, 2:3] - rowb[2:3, :]
    r2 = dx * dx + dy * dy + dz * dz
    qq = colb[:, 3:4] * rowb[3:4, :]

    il = jax.lax.broadcasted_iota(jnp.int32, (BI, BJ), 0)
    jl = jax.lax.broadcasted_iota(jnp.int32, (BI, BJ), 1)
    mask = ((jl > il) | (bi != bj)) & (r2 > 0.0)

    qqm = jnp.where(mask, qq, 0.0)
    r2s = jnp.where(mask, r2, 1.0)
    rinv = jax.lax.rsqrt(r2s)
    r = r2s * rinv
    a = jnp.clip(r - R_ON, 0.0, 1.0) * (1.0 / (R_OFF - R_ON))
    fs = 1.0 / (1.0 + jnp.exp(1.0 / (1.0 - a) - 1.0 / a))
    g = jax.lax.rsqrt(r2s + 1.0)
    e = qqm * (fs * (g - rinv) + rinv)
    esum = jnp.sum(e, axis=(0, 1), keepdims=True)

    @pl.when(t == 0)
    def _():
        out_ref[...] = jnp.zeros_like(out_ref)

    out_ref[...] += KE_KCAL * esum


@jax.jit
def kernel(q, xyz):
    n = xyz.shape[0]
    row = jnp.concatenate([xyz.T, q[None, :], jnp.zeros((4, n), jnp.float32)], axis=0)
    col = row.T

    grid_spec = pltpu.PrefetchScalarGridSpec(
        num_scalar_prefetch=2,
        grid=(_T,),
        in_specs=[
            pl.BlockSpec((BI, 8), lambda t, bis, bjs: (bis[t], 0)),
            pl.BlockSpec((8, BJ), lambda t, bis, bjs: (0, bjs[t])),
        ],
        out_specs=pl.BlockSpec((1, 1), lambda t, bis, bjs: (0, 0)),
    )
    out = pl.pallas_call(
        _tile_kernel,
        grid_spec=grid_spec,
        out_shape=jax.ShapeDtypeStruct((1, 1), jnp.float32),
        compiler_params=pltpu.CompilerParams(
            dimension_semantics=("arbitrary",),
        ),
    )(jnp.asarray(_BIS), jnp.asarray(_BJS), col, row)
    return out[0, 0]


# speedup vs baseline: 2.2853x; 1.5413x over previous
import functools

import jax
import jax.numpy as jnp
import numpy as np
from jax.experimental import pallas as pl
from jax.experimental.pallas import tpu as pltpu

KE_KCAL = 332.0716
R_ON = 4.0
R_OFF = 5.0

N = 2048
BI = 512
BJ = 512
_NB = N // BI
_BIS = np.array([i for i in range(_NB) for j in range(i, _NB)], dtype=np.int32)
_BJS = np.array([j for i in range(_NB) for j in range(i, _NB)], dtype=np.int32)
_T = len(_BIS)


def _tile_kernel(bis_ref, bjs_ref, col_ref, row_ref, out_ref):
    t = pl.program_id(0)
    bi = bis_ref[t]
    bj = bjs_ref[t]

    colb = col_ref[...]
    rowb = row_ref[...]

    dx = colb[:, 0:1] - rowb[0:1, :]
    dy = colb[:, 1:2] - rowb[1:2, :]
    dz = colb[:, 2:3] - rowb[2:3, :]
    r2 = dx * dx + dy * dy + dz * dz
    qq = colb[:, 3:4] * rowb[3:4, :]

    il = jax.lax.broadcasted_iota(jnp.int32, (BI, BJ), 0)
    jl = jax.lax.broadcasted_iota(jnp.int32, (BI, BJ), 1)
    mask = ((jl > il) | (bi != bj)) & (r2 > 0.0)

    qqm = jnp.where(mask, qq, 0.0)
    r2s = jnp.where(mask, r2, 1.0)
    rinv = jax.lax.rsqrt(r2s)
    r = r2s * rinv
    a = jnp.clip(r - R_ON, 0.0, 1.0) * (1.0 / (R_OFF - R_ON))
    fs = 1.0 / (1.0 + jnp.exp(1.0 / (1.0 - a) - 1.0 / a))
    g = jax.lax.rsqrt(r2s + 1.0)
    e = qqm * (fs * (g - rinv) + rinv)
    esum = jnp.sum(e, axis=(0, 1), keepdims=True)

    @pl.when(t == 0)
    def _():
        out_ref[...] = jnp.zeros_like(out_ref)

    out_ref[...] += KE_KCAL * esum


@jax.jit
def kernel(q, xyz):
    n = xyz.shape[0]
    row = jnp.concatenate([xyz.T, q[None, :], jnp.zeros((4, n), jnp.float32)], axis=0)
    col = row.T

    grid_spec = pltpu.PrefetchScalarGridSpec(
        num_scalar_prefetch=2,
        grid=(_T,),
        in_specs=[
            pl.BlockSpec((BI, 8), lambda t, bis, bjs: (bis[t], 0)),
            pl.BlockSpec((8, BJ), lambda t, bis, bjs: (0, bjs[t])),
        ],
        out_specs=pl.BlockSpec((1, 1), lambda t, bis, bjs: (0, 0)),
    )
    out = pl.pallas_call(
        _tile_kernel,
        grid_spec=grid_spec,
        out_shape=jax.ShapeDtypeStruct((1, 1), jnp.float32),
        compiler_params=pltpu.CompilerParams(
            dimension_semantics=("arbitrary",),
        ),
    )(jnp.asarray(_BIS), jnp.asarray(_BJS), col, row)
    return out[0, 0]
